# baseline (device time: 12915 ns/iter reference)
import jax
import jax.numpy as jnp
from jax import lax
from jax.experimental import pallas as pl
from jax.experimental.pallas import tpu as pltpu

N_DEV = 4
DISTS = (2, 1, 3)


def kernel(x, w_mat):
    m_per, k = x.shape
    _, n = w_mat.shape
    n_per = n // N_DEV

    def body(x_hbm, w_hbm, out_ref, x_v, w_v, send_buf, recv_buf,
             send_sems, recv_sems, ready_sems, in_sems):
        my = lax.axis_index("i")

        barrier_sem = pltpu.get_barrier_semaphore()
        pl.semaphore_signal(
            barrier_sem, inc=1,
            device_id=(my,), device_id_type=pl.DeviceIdType.MESH,
        )
        pl.semaphore_wait(barrier_sem, 1)

        cx = pltpu.make_async_copy(x_hbm, x_v, in_sems.at[0])
        cw = pltpu.make_async_copy(w_hbm, w_v, in_sems.at[1])
        cx.start()
        cw.start()

        for i, d in enumerate(DISTS):
            pl.semaphore_signal(
                ready_sems.at[i], inc=1,
                device_id=((my - d) % N_DEV,),
                device_id_type=pl.DeviceIdType.MESH,
            )

        cx.wait()
        cw.wait()
        rdmas = []
        for i, d in enumerate(DISTS):
            tgt = (my + d) % N_DEV
            yc = jnp.dot(
                x_v[:, :], w_v[:, pl.ds(tgt * n_per, n_per)],
                preferred_element_type=jnp.float32,
            )
            send_buf[i, :, :] = (yc * jax.nn.sigmoid(yc)).astype(jnp.bfloat16)
            pl.semaphore_wait(ready_sems.at[i], 1)
            rdma = pltpu.make_async_remote_copy(
                src_ref=send_buf.at[i],
                dst_ref=recv_buf.at[i],
                send_sem=send_sems.at[i],
                recv_sem=recv_sems.at[i],
                device_id=(tgt,),
                device_id_type=pl.DeviceIdType.MESH,
            )
            rdma.start()
            rdmas.append(rdma)

        yc = jnp.dot(
            x_v[:, :], w_v[:, pl.ds(my * n_per, n_per)],
            preferred_element_type=jnp.float32,
        )
        out_ref[pl.ds(my * m_per, m_per), :] = yc * jax.nn.sigmoid(yc)

        for i in (1, 2, 0):
            rdmas[i].wait_recv()
            src = (my - DISTS[i]) % N_DEV
            out_ref[pl.ds(src * m_per, m_per), :] = recv_buf[i, :, :].astype(
                jnp.float32
            )
        for rdma in rdmas:
            rdma.wait_send()

    out_shape = jax.ShapeDtypeStruct((N_DEV * m_per, n_per), jnp.float32)
    return pl.pallas_call(
        body,
        out_shape=out_shape,
        in_specs=[
            pl.BlockSpec(memory_space=pl.ANY),
            pl.BlockSpec(memory_space=pl.ANY),
        ],
        out_specs=pl.BlockSpec(memory_space=pltpu.VMEM),
        scratch_shapes=[
            pltpu.VMEM((m_per, k), jnp.float32),
            pltpu.VMEM((k, n), jnp.float32),
            pltpu.VMEM((N_DEV - 1, m_per, n_per), jnp.bfloat16),
            pltpu.VMEM((N_DEV - 1, m_per, n_per), jnp.bfloat16),
            pltpu.SemaphoreType.DMA((N_DEV - 1,)),
            pltpu.SemaphoreType.DMA((N_DEV - 1,)),
            pltpu.SemaphoreType.REGULAR((N_DEV - 1,)),
            pltpu.SemaphoreType.DMA((2,)),
        ],
        compiler_params=pltpu.CompilerParams(collective_id=0),
    )(x, w_mat)


# device time: 12357 ns/iter; 1.0452x vs baseline; 1.0452x over previous
import jax
import jax.numpy as jnp
from jax import lax
from jax.experimental import pallas as pl
from jax.experimental.pallas import tpu as pltpu

N_DEV = 4
DISTS = (2, 1, 3)


def kernel(x, w_mat):
    m_per, k = x.shape
    _, n = w_mat.shape
    n_per = n // N_DEV

    def body(x_ref, w_ref, out_ref, send_buf, recv_buf, send_sems, recv_sems,
             ready_sems):
        my = lax.axis_index("i")

        barrier_sem = pltpu.get_barrier_semaphore()
        pl.semaphore_signal(
            barrier_sem, inc=1,
            device_id=(my,), device_id_type=pl.DeviceIdType.MESH,
        )
        pl.semaphore_wait(barrier_sem, 1)

        for i, d in enumerate(DISTS):
            pl.semaphore_signal(
                ready_sems.at[i], inc=1,
                device_id=((my - d) % N_DEV,),
                device_id_type=pl.DeviceIdType.MESH,
            )

        rdmas = []
        for i, d in enumerate(DISTS):
            tgt = (my + d) % N_DEV
            yc = jnp.dot(
                x_ref[:, :], w_ref[:, pl.ds(tgt * n_per, n_per)],
                preferred_element_type=jnp.float32,
            )
            send_buf[i, :, :] = (yc * jax.nn.sigmoid(yc)).astype(jnp.bfloat16)
            pl.semaphore_wait(ready_sems.at[i], 1)
            rdma = pltpu.make_async_remote_copy(
                src_ref=send_buf.at[i],
                dst_ref=recv_buf.at[i],
                send_sem=send_sems.at[i],
                recv_sem=recv_sems.at[i],
                device_id=(tgt,),
                device_id_type=pl.DeviceIdType.MESH,
            )
            rdma.start()
            rdmas.append(rdma)

        yc = jnp.dot(
            x_ref[:, :], w_ref[:, pl.ds(my * n_per, n_per)],
            preferred_element_type=jnp.float32,
        )
        out_ref[pl.ds(my * m_per, m_per), :] = yc * jax.nn.sigmoid(yc)

        for i in (1, 2, 0):
            rdmas[i].wait_recv()
            src = (my - DISTS[i]) % N_DEV
            out_ref[pl.ds(src * m_per, m_per), :] = recv_buf[i, :, :].astype(
                jnp.float32
            )
        for rdma in rdmas:
            rdma.wait_send()

    out_shape = jax.ShapeDtypeStruct((N_DEV * m_per, n_per), jnp.float32)
    return pl.pallas_call(
        body,
        out_shape=out_shape,
        in_specs=[
            pl.BlockSpec(memory_space=pltpu.VMEM),
            pl.BlockSpec(memory_space=pltpu.VMEM),
        ],
        out_specs=pl.BlockSpec(memory_space=pltpu.VMEM),
        scratch_shapes=[
            pltpu.VMEM((N_DEV - 1, m_per, n_per), jnp.bfloat16),
            pltpu.VMEM((N_DEV - 1, m_per, n_per), jnp.bfloat16),
            pltpu.SemaphoreType.DMA((N_DEV - 1,)),
            pltpu.SemaphoreType.DMA((N_DEV - 1,)),
            pltpu.SemaphoreType.REGULAR((N_DEV - 1,)),
        ],
        compiler_params=pltpu.CompilerParams(collective_id=0),
    )(x, w_mat)
